# baseline (speedup 1.0000x reference)
import jax
import jax.numpy as jnp
from jax import lax
from jax.experimental import pallas as pl
from jax.experimental.pallas import tpu as pltpu
from jax.experimental.pallas import tpu_sc as plsc

N = 10000
F = 128
FH = 64
H2 = 256

NC = 2
NS = 16
NW = NC * NS

GW = 128
G_ALL = 2560
E_PAD = G_ALL * GW
G_HALF = G_ALL // 2
GE_F = G_HALF // NW
GE = G_ALL // NW
N_PAD = 10240
RT = N_PAD // NS

_MESH = plsc.VectorSubcoreMesh(core_axis_name="c", subcore_axis_name="s",
                               num_cores=NC, num_subcores=NS)


def _agg_pipeline(x_view, src_v, dst_v, rows, sems, acc_sh, ngroups):
  pltpu.async_copy(x_view.at[src_v.at[0]], rows[0], sems[0])

  @pl.loop(0, ngroups, step=2)
  def _(g0):
    for b in range(2):
      g = g0 + b

      @pl.when(g + 1 < ngroups)
      def _():
        pltpu.async_copy(x_view.at[src_v.at[g + 1]], rows[1 - b],
                         sems[1 - b])

      pltpu.make_async_copy(x_view.at[src_v.at[g]], rows[b], sems[b]).wait()
      pltpu.sync_copy(rows[b], acc_sh.at[dst_v.at[g]], add=True)


ZB = 64


def _edgesplit_body(x_hbm, src_hbm, dst_hbm, zacc_hbm, out_hbm,
                    src_v, dst_v, rows0, rows1, acc_sh, sem0, sem1):
  c = lax.axis_index("c")
  s = lax.axis_index("s")
  wid = c * NS + s

  @pl.loop(0, RT // ZB)
  def _(i):
    pltpu.sync_copy(zacc_hbm, acc_sh.at[pl.ds(s * RT + i * ZB, ZB)])

  pltpu.sync_copy(src_hbm.at[pl.ds(wid * GE_F, GE_F)], src_v)
  pltpu.sync_copy(dst_hbm.at[pl.ds(wid * GE_F, GE_F)], dst_v)
  plsc.subcore_barrier()

  _agg_pipeline(x_hbm, src_v, dst_v, (rows0, rows1), (sem0, sem1),
                acc_sh, GE_F)

  plsc.subcore_barrier()
  pltpu.sync_copy(acc_sh.at[pl.ds(s * RT, RT)],
                  out_hbm.at[c].at[pl.ds(s * RT, RT)])


_sc_agg_full = pl.kernel(
    _edgesplit_body,
    out_type=jax.ShapeDtypeStruct((NC, N_PAD, F), jnp.float32),
    mesh=_MESH,
    compiler_params=pltpu.CompilerParams(use_tc_tiling_on_sc=False),
    scratch_types=[
        pltpu.VMEM((GE_F, GW), jnp.int32),
        pltpu.VMEM((GE_F, GW), jnp.int32),
        pltpu.VMEM((GW, F), jnp.float32),
        pltpu.VMEM((GW, F), jnp.float32),
        pltpu.VMEM_SHARED((N_PAD, F), jnp.float32),
        pltpu.SemaphoreType.DMA,
        pltpu.SemaphoreType.DMA,
    ],
)


def _edgesplit8_body(x_hbm, src_hbm, dst_hbm, zacc_hbm, out_hbm,
                     src_v, dst_v, rows0, rows1, acc_sh, sem0, sem1):
  c = lax.axis_index("c")
  s = lax.axis_index("s")
  wid = c * NS + s

  pltpu.sync_copy(zacc_hbm, acc_sh.at[pl.ds(s * RT, RT)])
  pltpu.sync_copy(src_hbm.at[pl.ds(wid * GE, GE)], src_v)
  pltpu.sync_copy(dst_hbm.at[pl.ds(wid * GE, GE)], dst_v)
  plsc.subcore_barrier()

  _agg_pipeline(x_hbm, src_v, dst_v, (rows0, rows1), (sem0, sem1),
                acc_sh, GE)

  plsc.subcore_barrier()
  pltpu.sync_copy(acc_sh.at[pl.ds(s * RT, RT)],
                  out_hbm.at[c].at[pl.ds(s * RT, RT)])


_sc_agg_8 = pl.kernel(
    _edgesplit8_body,
    out_type=jax.ShapeDtypeStruct((NC, N_PAD, 8), jnp.float32),
    mesh=_MESH,
    compiler_params=pltpu.CompilerParams(use_tc_tiling_on_sc=False),
    scratch_types=[
        pltpu.VMEM((GE, GW), jnp.int32),
        pltpu.VMEM((GE, GW), jnp.int32),
        pltpu.VMEM((GW, 8), jnp.float32),
        pltpu.VMEM((GW, 8), jnp.float32),
        pltpu.VMEM_SHARED((N_PAD, 8), jnp.float32),
        pltpu.SemaphoreType.DMA,
        pltpu.SemaphoreType.DMA,
    ],
)


def _counts_body(dst_hbm, zacc_hbm, ones_hbm, out_hbm,
                 dst_v, ones_v, cnt_sh):
  c = lax.axis_index("c")
  s = lax.axis_index("s")
  wid = c * NS + s

  pltpu.sync_copy(zacc_hbm, cnt_sh.at[pl.ds(s * RT, RT)])
  pltpu.sync_copy(dst_hbm.at[pl.ds(wid * GE, GE)], dst_v)
  pltpu.sync_copy(ones_hbm, ones_v)
  plsc.subcore_barrier()

  @pl.loop(0, GE)
  def _(g):
    pltpu.sync_copy(ones_v, cnt_sh.at[dst_v.at[g]], add=True)

  plsc.subcore_barrier()
  pltpu.sync_copy(cnt_sh.at[pl.ds(s * RT, RT)],
                  out_hbm.at[c].at[pl.ds(s * RT, RT)])


_sc_counts = pl.kernel(
    _counts_body,
    out_type=jax.ShapeDtypeStruct((NC, N_PAD, 8), jnp.float32),
    mesh=_MESH,
    compiler_params=pltpu.CompilerParams(use_tc_tiling_on_sc=False),
    scratch_types=[
        pltpu.VMEM((GE, GW), jnp.int32),
        pltpu.VMEM((GW, 8), jnp.float32),
        pltpu.VMEM_SHARED((N_PAD, 8), jnp.float32),
    ],
)


BR = 400


def _mean(agg, cnt_ref):
  deg = (cnt_ref[0] + cnt_ref[1])[:, 0:1]
  return agg * (1.0 / jnp.maximum(deg, 1.0))


def _tc_layer1(agga_ref, aggb_ref, cnt_ref, x_ref, wl_ref, wr_ref, b_ref,
               o_ref):
  mean = _mean(agga_ref[0] + agga_ref[1] + aggb_ref[0] + aggb_ref[1],
               cnt_ref)
  h = jnp.dot(mean, wl_ref[...], preferred_element_type=jnp.float32)
  h = h + b_ref[...] + jnp.dot(x_ref[...], wr_ref[...],
                               preferred_element_type=jnp.float32)
  o_ref[...] = jnp.maximum(h, 0.0)


def _tc_layer2(agga_ref, aggb_ref, cnt_ref, h1_ref, wl_ref, wr_ref, b_ref,
               w3l_ref, h2_ref, z_ref):
  mean = _mean(agga_ref[0] + agga_ref[1] + aggb_ref[0] + aggb_ref[1],
               cnt_ref)
  h = jnp.dot(mean, wl_ref[...], preferred_element_type=jnp.float32)
  h = h + b_ref[...] + jnp.dot(h1_ref[...], wr_ref[...],
                               preferred_element_type=jnp.float32)
  h2 = jnp.maximum(h, 0.0)
  h2_ref[...] = h2
  z_ref[...] = jnp.dot(h2, w3l_ref[...], preferred_element_type=jnp.float32)


def _tc_layer3(part_ref, cnt_ref, h2_ref, w3r_ref, b3_ref, o_ref):
  agg = part_ref[0] + part_ref[1]
  mean = _mean(agg, cnt_ref)
  r = jnp.dot(h2_ref[...], w3r_ref[...], preferred_element_type=jnp.float32)
  logits = (mean + r + b3_ref[...])[:, 0:2]
  m = jnp.max(logits, axis=1, keepdims=True)
  lse = m + jnp.log(jnp.sum(jnp.exp(logits - m), axis=1, keepdims=True))
  o_ref[...] = logits - lse


def _row_spec(d):
  return pl.BlockSpec((BR, d), lambda i: (i, 0))


def _part_spec(d):
  return pl.BlockSpec((NC, BR, d), lambda i: (0, i, 0))


def _full(shape):
  return pl.BlockSpec(shape, lambda i: tuple(0 for _ in shape))


_GRID = (N // BR,)


def kernel(x, edge_index, W1_l, W1_r, b1, W2_l, W2_r, b2, W3_l, W3_r, b3):
  src = edge_index[0]
  dst = edge_index[1]
  npad = E_PAD - src.shape[0]
  src_p = jnp.concatenate([src, jnp.zeros((npad,), jnp.int32)])
  dst_p = jnp.concatenate(
      [dst, N + (jnp.arange(npad, dtype=jnp.int32) % (N_PAD - N))])
  src2d = src_p.reshape(G_ALL, GW)
  dst2d = dst_p.reshape(G_ALL, GW)

  zacc128 = jnp.zeros((ZB, F), jnp.float32)
  zacc8 = jnp.zeros((RT, 8), jnp.float32)
  ones = jnp.ones((GW, 8), jnp.float32)

  cnt = _sc_counts(dst2d, zacc8, ones)

  src_a, src_b = src2d[:G_HALF], src2d[G_HALF:]
  dst_a, dst_b = dst2d[:G_HALF], dst2d[G_HALF:]
  agg1a = _sc_agg_full(x, src_a, dst_a, zacc128)
  agg1b = _sc_agg_full(x, src_b, dst_b, zacc128)

  b1r = b1.reshape(1, F)
  h1 = pl.pallas_call(
      _tc_layer1,
      grid=_GRID,
      in_specs=[_part_spec(F), _part_spec(F), _part_spec(8), _row_spec(F),
                _full((F, F)), _full((F, F)), _full((1, F))],
      out_specs=_row_spec(F),
      out_shape=jax.ShapeDtypeStruct((N, F), jnp.float32),
  )(agg1a, agg1b, cnt, x, W1_l, W1_r, b1r)

  agg2a = _sc_agg_full(h1, src_a, dst_a, zacc128)
  agg2b = _sc_agg_full(h1, src_b, dst_b, zacc128)

  b2r = b2.reshape(1, H2)
  w3l_pad = jnp.pad(W3_l, ((0, 0), (0, 6)))
  h2, z = pl.pallas_call(
      _tc_layer2,
      grid=_GRID,
      in_specs=[_part_spec(F), _part_spec(F), _part_spec(8), _row_spec(F),
                _full((F, H2)), _full((F, H2)), _full((1, H2)),
                _full((H2, 8))],
      out_specs=[_row_spec(H2), _row_spec(8)],
      out_shape=[jax.ShapeDtypeStruct((N, H2), jnp.float32),
                 jax.ShapeDtypeStruct((N, 8), jnp.float32)],
  )(agg2a, agg2b, cnt, h1, W2_l, W2_r, b2r, w3l_pad)

  part3 = _sc_agg_8(z, src2d, dst2d, zacc8)

  w3r_pad = jnp.pad(W3_r, ((0, 0), (0, 6)))
  b3_pad = jnp.pad(b3, (0, 6)).reshape(1, 8)
  out = pl.pallas_call(
      _tc_layer3,
      grid=_GRID,
      in_specs=[_part_spec(8), _part_spec(8), _row_spec(H2),
                _full((H2, 8)), _full((1, 8))],
      out_specs=pl.BlockSpec((BR, 2), lambda i: (i, 0)),
      out_shape=jax.ShapeDtypeStruct((N, 2), jnp.float32),
  )(part3, cnt, h2, w3r_pad, b3_pad)
  return out


# speedup vs baseline: 1.0696x; 1.0696x over previous
import jax
import jax.numpy as jnp
from jax import lax
from jax.experimental import pallas as pl
from jax.experimental.pallas import tpu as pltpu
from jax.experimental.pallas import tpu_sc as plsc

N = 10000
F = 128
FH = 64
H2 = 256

NC = 2
NS = 16
NW = NC * NS

GW = 128
G_ALL = 2560
E_PAD = G_ALL * GW
GE = G_ALL // NW
N_PAD = 10240
RT = N_PAD // NS

_MESH = plsc.VectorSubcoreMesh(core_axis_name="c", subcore_axis_name="s",
                               num_cores=NC, num_subcores=NS)


def _agg_pipeline(x_view, src_v, dst_v, rows, sems, acc_sh, ngroups):
  pltpu.async_copy(x_view.at[src_v.at[0]], rows[0], sems[0])

  @pl.loop(0, ngroups, step=2)
  def _(g0):
    for b in range(2):
      g = g0 + b

      @pl.when(g + 1 < ngroups)
      def _():
        pltpu.async_copy(x_view.at[src_v.at[g + 1]], rows[1 - b],
                         sems[1 - b])

      pltpu.make_async_copy(x_view.at[src_v.at[g]], rows[b], sems[b]).wait()
      pltpu.sync_copy(rows[b], acc_sh.at[dst_v.at[g]], add=True)


ZB = 64


def _edgesplit_body(x_hbm, pk_hbm, zacc_hbm, out_hbm,
                    pk_v, srcb, dstb, rows0, rows1, acc_sh, sem0, sem1):
  c = lax.axis_index("c")
  s = lax.axis_index("s")
  wid = c * NS + s

  @pl.loop(0, RT // ZB)
  def _(i):
    pltpu.sync_copy(zacc_hbm, acc_sh.at[pl.ds(s * RT + i * ZB, ZB)])

  pltpu.sync_copy(pk_hbm.at[pl.ds(wid * GE, GE)], pk_v)
  plsc.subcore_barrier()

  def unpack(g, b):
    for k in range(GW // 16):
      v = pk_v[g, pl.ds(k * 16, 16)]
      srcb[b, pl.ds(k * 16, 16)] = lax.bitwise_and(v, 0xFFFF)
      dstb[b, pl.ds(k * 16, 16)] = lax.shift_right_logical(v, 16)

  rows = (rows0, rows1)
  sems = (sem0, sem1)
  unpack(0, 0)
  pltpu.async_copy(x_hbm.at[srcb.at[0]], rows[0], sems[0])

  @pl.loop(0, GE, step=2)
  def _(g0):
    for b in range(2):
      g = g0 + b

      @pl.when(g + 1 < GE)
      def _():
        unpack(g + 1, 1 - b)
        pltpu.async_copy(x_hbm.at[srcb.at[1 - b]], rows[1 - b],
                         sems[1 - b])

      pltpu.make_async_copy(x_hbm.at[srcb.at[b]], rows[b], sems[b]).wait()
      pltpu.sync_copy(rows[b], acc_sh.at[dstb.at[b]], add=True)

  plsc.subcore_barrier()
  pltpu.sync_copy(acc_sh.at[pl.ds(s * RT, RT)],
                  out_hbm.at[c].at[pl.ds(s * RT, RT)])


_sc_agg_full = pl.kernel(
    _edgesplit_body,
    out_type=jax.ShapeDtypeStruct((NC, N_PAD, F), jnp.float32),
    mesh=_MESH,
    compiler_params=pltpu.CompilerParams(use_tc_tiling_on_sc=False),
    scratch_types=[
        pltpu.VMEM((GE, GW), jnp.int32),
        pltpu.VMEM((2, GW), jnp.int32),
        pltpu.VMEM((2, GW), jnp.int32),
        pltpu.VMEM((GW, F), jnp.float32),
        pltpu.VMEM((GW, F), jnp.float32),
        pltpu.VMEM_SHARED((N_PAD, F), jnp.float32),
        pltpu.SemaphoreType.DMA,
        pltpu.SemaphoreType.DMA,
    ],
)


def _edgesplit16_body(x_hbm, src_hbm, dst_hbm, zacc_hbm, out_hbm,
                      src_v, dst_v, rows0, rows1, acc_sh, sem0, sem1):
  c = lax.axis_index("c")
  s = lax.axis_index("s")
  wid = c * NS + s

  pltpu.sync_copy(zacc_hbm, acc_sh.at[pl.ds(s * RT, RT)])
  pltpu.sync_copy(src_hbm.at[pl.ds(wid * GE, GE)], src_v)
  pltpu.sync_copy(dst_hbm.at[pl.ds(wid * GE, GE)], dst_v)
  plsc.subcore_barrier()

  _agg_pipeline(x_hbm, src_v, dst_v, (rows0, rows1), (sem0, sem1),
                acc_sh, GE)

  plsc.subcore_barrier()
  pltpu.sync_copy(acc_sh.at[pl.ds(s * RT, RT)],
                  out_hbm.at[c].at[pl.ds(s * RT, RT)])


_sc_agg_16 = pl.kernel(
    _edgesplit16_body,
    out_type=jax.ShapeDtypeStruct((NC, N_PAD, 16), jnp.float32),
    mesh=_MESH,
    compiler_params=pltpu.CompilerParams(use_tc_tiling_on_sc=False),
    scratch_types=[
        pltpu.VMEM((GE, GW), jnp.int32),
        pltpu.VMEM((GE, GW), jnp.int32),
        pltpu.VMEM((GW, 16), jnp.float32),
        pltpu.VMEM((GW, 16), jnp.float32),
        pltpu.VMEM_SHARED((N_PAD, 16), jnp.float32),
        pltpu.SemaphoreType.DMA,
        pltpu.SemaphoreType.DMA,
    ],
)


def _counts_body(dst_hbm, zacc_hbm, ones_hbm, out_hbm,
                 dst_v, ones_v, cnt_sh):
  c = lax.axis_index("c")
  s = lax.axis_index("s")
  wid = c * NS + s

  pltpu.sync_copy(zacc_hbm, cnt_sh.at[pl.ds(s * RT, RT)])
  pltpu.sync_copy(dst_hbm.at[pl.ds(wid * GE, GE)], dst_v)
  pltpu.sync_copy(ones_hbm, ones_v)
  plsc.subcore_barrier()

  @pl.loop(0, GE)
  def _(g):
    pltpu.sync_copy(ones_v, cnt_sh.at[dst_v.at[g]], add=True)

  plsc.subcore_barrier()
  pltpu.sync_copy(cnt_sh.at[pl.ds(s * RT, RT)],
                  out_hbm.at[c].at[pl.ds(s * RT, RT)])


_sc_counts = pl.kernel(
    _counts_body,
    out_type=jax.ShapeDtypeStruct((NC, N_PAD, 16), jnp.float32),
    mesh=_MESH,
    compiler_params=pltpu.CompilerParams(use_tc_tiling_on_sc=False),
    scratch_types=[
        pltpu.VMEM((GE, GW), jnp.int32),
        pltpu.VMEM((GW, 16), jnp.float32),
        pltpu.VMEM_SHARED((N_PAD, 16), jnp.float32),
    ],
)


BR = 400


def _mean(agg, cnt_ref):
  deg = (cnt_ref[0] + cnt_ref[1])[:, 0:1]
  return agg * (1.0 / jnp.maximum(deg, 1.0))


def _tc_layer1(agg_ref, cnt_ref, x_ref, wl_ref, wr_ref, b_ref, o_ref):
  mean = _mean(agg_ref[0] + agg_ref[1], cnt_ref)
  h = jnp.dot(mean, wl_ref[...], preferred_element_type=jnp.float32)
  h = h + b_ref[...] + jnp.dot(x_ref[...], wr_ref[...],
                               preferred_element_type=jnp.float32)
  o_ref[...] = jnp.maximum(h, 0.0)


def _tc_layer2(agg_ref, cnt_ref, h1_ref, wl_ref, wr_ref, b_ref,
               w3l_ref, h2_ref, z_ref):
  mean = _mean(agg_ref[0] + agg_ref[1], cnt_ref)
  h = jnp.dot(mean, wl_ref[...], preferred_element_type=jnp.float32)
  h = h + b_ref[...] + jnp.dot(h1_ref[...], wr_ref[...],
                               preferred_element_type=jnp.float32)
  h2 = jnp.maximum(h, 0.0)
  h2_ref[...] = h2
  z_ref[...] = jnp.dot(h2, w3l_ref[...], preferred_element_type=jnp.float32)


def _tc_layer3(part_ref, cnt_ref, h2_ref, w3r_ref, b3_ref, o_ref):
  agg = part_ref[0] + part_ref[1]
  mean = _mean(agg, cnt_ref)
  r = jnp.dot(h2_ref[...], w3r_ref[...], preferred_element_type=jnp.float32)
  logits = (mean + r + b3_ref[...])[:, 0:2]
  m = jnp.max(logits, axis=1, keepdims=True)
  lse = m + jnp.log(jnp.sum(jnp.exp(logits - m), axis=1, keepdims=True))
  o_ref[...] = logits - lse


def _row_spec(d):
  return pl.BlockSpec((BR, d), lambda i: (i, 0))


def _part_spec(d):
  return pl.BlockSpec((NC, BR, d), lambda i: (0, i, 0))


def _full(shape):
  return pl.BlockSpec(shape, lambda i: tuple(0 for _ in shape))


_GRID = (N // BR,)


def kernel(x, edge_index, W1_l, W1_r, b1, W2_l, W2_r, b2, W3_l, W3_r, b3):
  src = edge_index[0]
  dst = edge_index[1]
  npad = E_PAD - src.shape[0]
  src_p = jnp.concatenate([src, jnp.zeros((npad,), jnp.int32)])
  dst_p = jnp.concatenate(
      [dst, N + (jnp.arange(npad, dtype=jnp.int32) % (N_PAD - N))])
  src2d = src_p.reshape(G_ALL, GW)
  dst2d = dst_p.reshape(G_ALL, GW)
  packed = jnp.bitwise_or(src2d, jnp.left_shift(dst2d, 16))

  zacc128 = jnp.zeros((ZB, F), jnp.float32)
  zacc16 = jnp.zeros((RT, 16), jnp.float32)
  ones = jnp.ones((GW, 16), jnp.float32)

  cnt = _sc_counts(dst2d, zacc16, ones)

  agg1 = _sc_agg_full(x, packed, zacc128)

  b1r = b1.reshape(1, F)
  h1 = pl.pallas_call(
      _tc_layer1,
      grid=_GRID,
      in_specs=[_part_spec(F), _part_spec(16), _row_spec(F),
                _full((F, F)), _full((F, F)), _full((1, F))],
      out_specs=_row_spec(F),
      out_shape=jax.ShapeDtypeStruct((N, F), jnp.float32),
  )(agg1, cnt, x, W1_l, W1_r, b1r)

  agg2 = _sc_agg_full(h1, packed, zacc128)

  b2r = b2.reshape(1, H2)
  w3l_pad = jnp.pad(W3_l, ((0, 0), (0, 14)))
  h2, z = pl.pallas_call(
      _tc_layer2,
      grid=_GRID,
      in_specs=[_part_spec(F), _part_spec(16), _row_spec(F),
                _full((F, H2)), _full((F, H2)), _full((1, H2)),
                _full((H2, 16))],
      out_specs=[_row_spec(H2), _row_spec(16)],
      out_shape=[jax.ShapeDtypeStruct((N, H2), jnp.float32),
                 jax.ShapeDtypeStruct((N, 16), jnp.float32)],
  )(agg2, cnt, h1, W2_l, W2_r, b2r, w3l_pad)

  part3 = _sc_agg_16(z, src2d, dst2d, zacc16)

  w3r_pad = jnp.pad(W3_r, ((0, 0), (0, 14)))
  b3_pad = jnp.pad(b3, (0, 14)).reshape(1, 16)
  out = pl.pallas_call(
      _tc_layer3,
      grid=_GRID,
      in_specs=[_part_spec(16), _part_spec(16), _row_spec(H2),
                _full((H2, 16)), _full((1, 16))],
      out_specs=pl.BlockSpec((BR, 2), lambda i: (i, 0)),
      out_shape=jax.ShapeDtypeStruct((N, 2), jnp.float32),
  )(part3, cnt, h2, w3r_pad, b3_pad)
  return out


# speedup vs baseline: 1.2972x; 1.2128x over previous
import jax
import jax.numpy as jnp
from jax import lax
from jax.experimental import pallas as pl
from jax.experimental.pallas import tpu as pltpu
from jax.experimental.pallas import tpu_sc as plsc

N = 10000
F = 128
FH = 64
H2 = 256

NC = 2
NS = 16
NW = NC * NS

GW = 128
G_ALL = 2560
E_PAD = G_ALL * GW
GT = G_ALL // NS
GE = G_ALL // NW
N_PAD = 10240
RT = N_PAD // NS
ZT = N // NS

_MESH = plsc.VectorSubcoreMesh(core_axis_name="c", subcore_axis_name="s",
                               num_cores=NC, num_subcores=NS)


def _colsplit_body(x2_hbm, pk_hbm, zacc_hbm, out_hbm,
                   pk_v, srcb, dstb, rows0, rows1, acc_sh, sem0, sem1):
  c = lax.axis_index("c")
  s = lax.axis_index("s")

  pltpu.sync_copy(zacc_hbm, acc_sh.at[pl.ds(s * RT, RT)])
  pltpu.sync_copy(pk_hbm.at[pl.ds(s * GT, GT)], pk_v)
  plsc.subcore_barrier()

  def unpack(g, b):
    for k in range(GW // 16):
      v = pk_v[g, pl.ds(k * 16, 16)]
      src = lax.bitwise_and(v, 0xFFFF)
      srcb[b, pl.ds(k * 16, 16)] = src + src + c
      dstb[b, pl.ds(k * 16, 16)] = lax.shift_right_logical(v, 16)

  rows = (rows0, rows1)
  sems = (sem0, sem1)
  unpack(0, 0)
  pltpu.async_copy(x2_hbm.at[srcb.at[0]], rows[0], sems[0])

  @pl.loop(0, GT, step=2)
  def _(g0):
    for b in range(2):
      g = g0 + b

      @pl.when(g + 1 < GT)
      def _():
        unpack(g + 1, 1 - b)
        pltpu.async_copy(x2_hbm.at[srcb.at[1 - b]], rows[1 - b],
                         sems[1 - b])

      pltpu.make_async_copy(x2_hbm.at[srcb.at[b]], rows[b], sems[b]).wait()
      pltpu.sync_copy(rows[b], acc_sh.at[dstb.at[b]], add=True)

  plsc.subcore_barrier()
  pltpu.sync_copy(acc_sh.at[pl.ds(s * RT, RT)],
                  out_hbm.at[c].at[pl.ds(s * RT, RT)])


_sc_agg_col = pl.kernel(
    _colsplit_body,
    out_type=jax.ShapeDtypeStruct((NC, N_PAD, FH), jnp.float32),
    mesh=_MESH,
    compiler_params=pltpu.CompilerParams(use_tc_tiling_on_sc=False),
    scratch_types=[
        pltpu.VMEM((GT, GW), jnp.int32),
        pltpu.VMEM((2, GW), jnp.int32),
        pltpu.VMEM((2, GW), jnp.int32),
        pltpu.VMEM((GW, FH), jnp.float32),
        pltpu.VMEM((GW, FH), jnp.float32),
        pltpu.VMEM_SHARED((N_PAD, FH), jnp.float32),
        pltpu.SemaphoreType.DMA,
        pltpu.SemaphoreType.DMA,
    ],
)


def _agg16_body(z_hbm, src_hbm, dst_hbm, zacc_hbm, out_hbm,
                src_v, dst_v, rows0, rows1, z_sh, acc_sh, sem0, sem1):
  c = lax.axis_index("c")
  s = lax.axis_index("s")
  wid = c * NS + s

  pltpu.sync_copy(z_hbm.at[pl.ds(s * ZT, ZT)], z_sh.at[pl.ds(s * ZT, ZT)])
  pltpu.sync_copy(zacc_hbm, acc_sh.at[pl.ds(s * RT, RT)])
  pltpu.sync_copy(src_hbm.at[pl.ds(wid * GE, GE)], src_v)
  pltpu.sync_copy(dst_hbm.at[pl.ds(wid * GE, GE)], dst_v)
  plsc.subcore_barrier()

  rows = (rows0, rows1)
  sems = (sem0, sem1)
  pltpu.async_copy(z_sh.at[src_v.at[0]], rows[0], sems[0])

  @pl.loop(0, GE, step=2)
  def _(g0):
    for b in range(2):
      g = g0 + b

      @pl.when(g + 1 < GE)
      def _():
        pltpu.async_copy(z_sh.at[src_v.at[g + 1]], rows[1 - b],
                         sems[1 - b])

      pltpu.make_async_copy(z_sh.at[src_v.at[g]], rows[b], sems[b]).wait()
      pltpu.sync_copy(rows[b], acc_sh.at[dst_v.at[g]], add=True)

  plsc.subcore_barrier()
  pltpu.sync_copy(acc_sh.at[pl.ds(s * RT, RT)],
                  out_hbm.at[c].at[pl.ds(s * RT, RT)])


_sc_agg_16 = pl.kernel(
    _agg16_body,
    out_type=jax.ShapeDtypeStruct((NC, N_PAD, 16), jnp.float32),
    mesh=_MESH,
    compiler_params=pltpu.CompilerParams(use_tc_tiling_on_sc=False),
    scratch_types=[
        pltpu.VMEM((GE, GW), jnp.int32),
        pltpu.VMEM((GE, GW), jnp.int32),
        pltpu.VMEM((GW, 16), jnp.float32),
        pltpu.VMEM((GW, 16), jnp.float32),
        pltpu.VMEM_SHARED((N, 16), jnp.float32),
        pltpu.VMEM_SHARED((N_PAD, 16), jnp.float32),
        pltpu.SemaphoreType.DMA,
        pltpu.SemaphoreType.DMA,
    ],
)


def _counts_body(dst_hbm, zacc_hbm, ones_hbm, out_hbm,
                 dst_v, ones_v, cnt_sh):
  c = lax.axis_index("c")
  s = lax.axis_index("s")
  wid = c * NS + s

  pltpu.sync_copy(zacc_hbm, cnt_sh.at[pl.ds(s * RT, RT)])
  pltpu.sync_copy(dst_hbm.at[pl.ds(wid * GE, GE)], dst_v)
  pltpu.sync_copy(ones_hbm, ones_v)
  plsc.subcore_barrier()

  @pl.loop(0, GE)
  def _(g):
    pltpu.sync_copy(ones_v, cnt_sh.at[dst_v.at[g]], add=True)

  plsc.subcore_barrier()
  pltpu.sync_copy(cnt_sh.at[pl.ds(s * RT, RT)],
                  out_hbm.at[c].at[pl.ds(s * RT, RT)])


_sc_counts = pl.kernel(
    _counts_body,
    out_type=jax.ShapeDtypeStruct((NC, N_PAD, 16), jnp.float32),
    mesh=_MESH,
    compiler_params=pltpu.CompilerParams(use_tc_tiling_on_sc=False),
    scratch_types=[
        pltpu.VMEM((GE, GW), jnp.int32),
        pltpu.VMEM((GW, 16), jnp.float32),
        pltpu.VMEM_SHARED((N_PAD, 16), jnp.float32),
    ],
)


BR = 400


def _mean(agg, cnt_ref):
  deg = (cnt_ref[0] + cnt_ref[1])[:, 0:1]
  return agg * (1.0 / jnp.maximum(deg, 1.0))


def _halves(ref):
  return jnp.concatenate([ref[0], ref[1]], axis=1)


def _tc_layer1(agg_ref, cnt_ref, x_ref, wl_ref, wr_ref, b_ref, o_ref):
  mean = _mean(_halves(agg_ref), cnt_ref)
  h = jnp.dot(mean, wl_ref[...], preferred_element_type=jnp.float32)
  h = h + b_ref[...] + jnp.dot(x_ref[...], wr_ref[...],
                               preferred_element_type=jnp.float32)
  o_ref[...] = jnp.maximum(h, 0.0)


def _tc_layer2(agg_ref, cnt_ref, h1_ref, wl_ref, wr_ref, b_ref, w3l_ref,
               h2_ref, z_ref):
  mean = _mean(_halves(agg_ref), cnt_ref)
  h = jnp.dot(mean, wl_ref[...], preferred_element_type=jnp.float32)
  h = h + b_ref[...] + jnp.dot(h1_ref[...], wr_ref[...],
                               preferred_element_type=jnp.float32)
  h2 = jnp.maximum(h, 0.0)
  h2_ref[...] = h2
  z_ref[...] = jnp.dot(h2, w3l_ref[...], preferred_element_type=jnp.float32)


def _tc_layer3(part_ref, cnt_ref, h2_ref, w3r_ref, b3_ref, o_ref):
  agg = part_ref[0] + part_ref[1]
  mean = _mean(agg, cnt_ref)
  r = jnp.dot(h2_ref[...], w3r_ref[...], preferred_element_type=jnp.float32)
  logits = (mean + r + b3_ref[...])[:, 0:2]
  m = jnp.max(logits, axis=1, keepdims=True)
  lse = m + jnp.log(jnp.sum(jnp.exp(logits - m), axis=1, keepdims=True))
  o_ref[...] = logits - lse


def _row_spec(d):
  return pl.BlockSpec((BR, d), lambda i: (i, 0))


def _part_spec(d):
  return pl.BlockSpec((NC, BR, d), lambda i: (0, i, 0))


def _full(shape):
  return pl.BlockSpec(shape, lambda i: tuple(0 for _ in shape))


_GRID = (N // BR,)


def kernel(x, edge_index, W1_l, W1_r, b1, W2_l, W2_r, b2, W3_l, W3_r, b3):
  src = edge_index[0]
  dst = edge_index[1]
  npad = E_PAD - src.shape[0]
  src_p = jnp.concatenate([src, jnp.zeros((npad,), jnp.int32)])
  dst_p = jnp.concatenate(
      [dst, N + (jnp.arange(npad, dtype=jnp.int32) % (N_PAD - N))])
  src2d = src_p.reshape(G_ALL, GW)
  dst2d = dst_p.reshape(G_ALL, GW)
  packed = jnp.bitwise_or(src2d, jnp.left_shift(dst2d, 16))

  zacc64 = jnp.zeros((RT, FH), jnp.float32)
  zacc16 = jnp.zeros((RT, 16), jnp.float32)
  ones = jnp.ones((GW, 16), jnp.float32)

  cnt = _sc_counts(dst2d, zacc16, ones)

  agg1 = _sc_agg_col(x.reshape(2 * N, FH), packed, zacc64)

  b1r = b1.reshape(1, F)
  h1 = pl.pallas_call(
      _tc_layer1,
      grid=_GRID,
      in_specs=[_part_spec(FH), _part_spec(16), _row_spec(F),
                _full((F, F)), _full((F, F)), _full((1, F))],
      out_specs=_row_spec(F),
      out_shape=jax.ShapeDtypeStruct((N, F), jnp.float32),
  )(agg1, cnt, x, W1_l, W1_r, b1r)

  agg2 = _sc_agg_col(h1.reshape(2 * N, FH), packed, zacc64)

  b2r = b2.reshape(1, H2)
  w3l_pad = jnp.pad(W3_l, ((0, 0), (0, 14)))
  h2, z = pl.pallas_call(
      _tc_layer2,
      grid=_GRID,
      in_specs=[_part_spec(FH), _part_spec(16), _row_spec(F),
                _full((F, H2)), _full((F, H2)), _full((1, H2)),
                _full((H2, 16))],
      out_specs=[_row_spec(H2), _row_spec(16)],
      out_shape=[jax.ShapeDtypeStruct((N, H2), jnp.float32),
                 jax.ShapeDtypeStruct((N, 16), jnp.float32)],
  )(agg2, cnt, h1, W2_l, W2_r, b2r, w3l_pad)

  part3 = _sc_agg_16(z, src2d, dst2d, zacc16)

  w3r_pad = jnp.pad(W3_r, ((0, 0), (0, 14)))
  b3_pad = jnp.pad(b3, (0, 14)).reshape(1, 16)
  out = pl.pallas_call(
      _tc_layer3,
      grid=_GRID,
      in_specs=[_part_spec(16), _part_spec(16), _row_spec(H2),
                _full((H2, 16)), _full((1, 16))],
      out_specs=pl.BlockSpec((BR, 2), lambda i: (i, 0)),
      out_shape=jax.ShapeDtypeStruct((N, 2), jnp.float32),
  )(part3, cnt, h2, w3r_pad, b3_pad)
  return out


# speedup vs baseline: 1.3016x; 1.0034x over previous
import jax
import jax.numpy as jnp
from jax import lax
from jax.experimental import pallas as pl
from jax.experimental.pallas import tpu as pltpu
from jax.experimental.pallas import tpu_sc as plsc

N = 10000
F = 128
FH = 64
H2 = 256

NC = 2
NS = 16
NW = NC * NS

GW = 128
G_ALL = 2560
E_PAD = G_ALL * GW
GT = G_ALL // NS
GE = G_ALL // NW
N_PAD = 10240
RT = N_PAD // NS
ZT = N // NS

_MESH = plsc.VectorSubcoreMesh(core_axis_name="c", subcore_axis_name="s",
                               num_cores=NC, num_subcores=NS)


def _colsplit_body(x2_hbm, src_hbm, dst_hbm, zacc_hbm, out_hbm,
                   src_v, dst_v, rows0, rows1, acc_sh, sem0, sem1):
  c = lax.axis_index("c")
  s = lax.axis_index("s")

  pltpu.sync_copy(zacc_hbm, acc_sh.at[pl.ds(s * RT, RT)])
  pltpu.sync_copy(src_hbm.at[c].at[pl.ds(s * GT, GT)], src_v)
  pltpu.sync_copy(dst_hbm.at[pl.ds(s * GT, GT)], dst_v)
  plsc.subcore_barrier()

  rows = (rows0, rows1)
  sems = (sem0, sem1)
  pltpu.async_copy(x2_hbm.at[src_v.at[0]], rows[0], sems[0])

  @pl.loop(0, GT, step=2)
  def _(g0):
    for b in range(2):
      g = g0 + b

      @pl.when(g + 1 < GT)
      def _():
        pltpu.async_copy(x2_hbm.at[src_v.at[g + 1]], rows[1 - b],
                         sems[1 - b])

      pltpu.make_async_copy(x2_hbm.at[src_v.at[g]], rows[b], sems[b]).wait()
      pltpu.sync_copy(rows[b], acc_sh.at[dst_v.at[g]], add=True)

  plsc.subcore_barrier()
  pltpu.sync_copy(acc_sh.at[pl.ds(s * RT, RT)],
                  out_hbm.at[c].at[pl.ds(s * RT, RT)])


_sc_agg_col = pl.kernel(
    _colsplit_body,
    out_type=jax.ShapeDtypeStruct((NC, N_PAD, FH), jnp.float32),
    mesh=_MESH,
    compiler_params=pltpu.CompilerParams(use_tc_tiling_on_sc=False),
    scratch_types=[
        pltpu.VMEM((GT, GW), jnp.int32),
        pltpu.VMEM((GT, GW), jnp.int32),
        pltpu.VMEM((GW, FH), jnp.float32),
        pltpu.VMEM((GW, FH), jnp.float32),
        pltpu.VMEM_SHARED((N_PAD, FH), jnp.float32),
        pltpu.SemaphoreType.DMA,
        pltpu.SemaphoreType.DMA,
    ],
)


def _agg16_body(z_hbm, src_hbm, dst_hbm, zacc_hbm, out_hbm,
                src_v, dst_v, rows0, rows1, z_sh, acc_sh, sem0, sem1):
  c = lax.axis_index("c")
  s = lax.axis_index("s")
  wid = c * NS + s

  pltpu.sync_copy(z_hbm.at[pl.ds(s * ZT, ZT)], z_sh.at[pl.ds(s * ZT, ZT)])
  pltpu.sync_copy(zacc_hbm, acc_sh.at[pl.ds(s * RT, RT)])
  pltpu.sync_copy(src_hbm.at[pl.ds(wid * GE, GE)], src_v)
  pltpu.sync_copy(dst_hbm.at[pl.ds(wid * GE, GE)], dst_v)
  plsc.subcore_barrier()

  rows = (rows0, rows1)
  sems = (sem0, sem1)
  pltpu.async_copy(z_sh.at[src_v.at[0]], rows[0], sems[0])

  @pl.loop(0, GE, step=2)
  def _(g0):
    for b in range(2):
      g = g0 + b

      @pl.when(g + 1 < GE)
      def _():
        pltpu.async_copy(z_sh.at[src_v.at[g + 1]], rows[1 - b],
                         sems[1 - b])

      pltpu.make_async_copy(z_sh.at[src_v.at[g]], rows[b], sems[b]).wait()
      pltpu.sync_copy(rows[b], acc_sh.at[dst_v.at[g]], add=True)

  plsc.subcore_barrier()
  pltpu.sync_copy(acc_sh.at[pl.ds(s * RT, RT)],
                  out_hbm.at[c].at[pl.ds(s * RT, RT)])


_sc_agg_16 = pl.kernel(
    _agg16_body,
    out_type=jax.ShapeDtypeStruct((NC, N_PAD, 16), jnp.float32),
    mesh=_MESH,
    compiler_params=pltpu.CompilerParams(use_tc_tiling_on_sc=False),
    scratch_types=[
        pltpu.VMEM((GE, GW), jnp.int32),
        pltpu.VMEM((GE, GW), jnp.int32),
        pltpu.VMEM((GW, 16), jnp.float32),
        pltpu.VMEM((GW, 16), jnp.float32),
        pltpu.VMEM_SHARED((N, 16), jnp.float32),
        pltpu.VMEM_SHARED((N_PAD, 16), jnp.float32),
        pltpu.SemaphoreType.DMA,
        pltpu.SemaphoreType.DMA,
    ],
)


def _counts_body(dst_hbm, zacc_hbm, ones_hbm, out_hbm,
                 dst_v, ones_v, cnt_sh):
  c = lax.axis_index("c")
  s = lax.axis_index("s")
  wid = c * NS + s

  pltpu.sync_copy(zacc_hbm, cnt_sh.at[pl.ds(s * RT, RT)])
  pltpu.sync_copy(dst_hbm.at[pl.ds(wid * GE, GE)], dst_v)
  pltpu.sync_copy(ones_hbm, ones_v)
  plsc.subcore_barrier()

  @pl.loop(0, GE)
  def _(g):
    pltpu.sync_copy(ones_v, cnt_sh.at[dst_v.at[g]], add=True)

  plsc.subcore_barrier()
  pltpu.sync_copy(cnt_sh.at[pl.ds(s * RT, RT)],
                  out_hbm.at[c].at[pl.ds(s * RT, RT)])


_sc_counts = pl.kernel(
    _counts_body,
    out_type=jax.ShapeDtypeStruct((NC, N_PAD, 16), jnp.float32),
    mesh=_MESH,
    compiler_params=pltpu.CompilerParams(use_tc_tiling_on_sc=False),
    scratch_types=[
        pltpu.VMEM((GE, GW), jnp.int32),
        pltpu.VMEM((GW, 16), jnp.float32),
        pltpu.VMEM_SHARED((N_PAD, 16), jnp.float32),
    ],
)


BR = 400


def _mean(agg, cnt_ref):
  deg = (cnt_ref[0] + cnt_ref[1])[:, 0:1]
  return agg * (1.0 / jnp.maximum(deg, 1.0))


def _halves(ref):
  return jnp.concatenate([ref[0], ref[1]], axis=1)


def _tc_layer1(agg_ref, cnt_ref, x_ref, wl_ref, wr_ref, b_ref, o_ref):
  mean = _mean(_halves(agg_ref), cnt_ref)
  h = jnp.dot(mean, wl_ref[...], preferred_element_type=jnp.float32)
  h = h + b_ref[...] + jnp.dot(x_ref[...], wr_ref[...],
                               preferred_element_type=jnp.float32)
  o_ref[...] = jnp.maximum(h, 0.0)


def _tc_layer2(agg_ref, cnt_ref, h1_ref, wl_ref, wr_ref, b_ref, w3l_ref,
               h2_ref, z_ref):
  mean = _mean(_halves(agg_ref), cnt_ref)
  h = jnp.dot(mean, wl_ref[...], preferred_element_type=jnp.float32)
  h = h + b_ref[...] + jnp.dot(h1_ref[...], wr_ref[...],
                               preferred_element_type=jnp.float32)
  h2 = jnp.maximum(h, 0.0)
  h2_ref[...] = h2
  z_ref[...] = jnp.dot(h2, w3l_ref[...], preferred_element_type=jnp.float32)


def _tc_layer3(part_ref, cnt_ref, h2_ref, w3r_ref, b3_ref, o_ref):
  agg = part_ref[0] + part_ref[1]
  mean = _mean(agg, cnt_ref)
  r = jnp.dot(h2_ref[...], w3r_ref[...], preferred_element_type=jnp.float32)
  logits = (mean + r + b3_ref[...])[:, 0:2]
  m = jnp.max(logits, axis=1, keepdims=True)
  lse = m + jnp.log(jnp.sum(jnp.exp(logits - m), axis=1, keepdims=True))
  o_ref[...] = logits - lse


def _row_spec(d):
  return pl.BlockSpec((BR, d), lambda i: (i, 0))


def _part_spec(d):
  return pl.BlockSpec((NC, BR, d), lambda i: (0, i, 0))


def _full(shape):
  return pl.BlockSpec(shape, lambda i: tuple(0 for _ in shape))


_GRID = (N // BR,)


def kernel(x, edge_index, W1_l, W1_r, b1, W2_l, W2_r, b2, W3_l, W3_r, b3):
  src = edge_index[0]
  dst = edge_index[1]
  npad = E_PAD - src.shape[0]
  src_p = jnp.concatenate([src, jnp.zeros((npad,), jnp.int32)])
  dst_p = jnp.concatenate(
      [dst, N + (jnp.arange(npad, dtype=jnp.int32) % (N_PAD - N))])
  src2d = src_p.reshape(G_ALL, GW)
  dst2d = dst_p.reshape(G_ALL, GW)
  src_off = jnp.stack([2 * src2d, 2 * src2d + 1])

  zacc64 = jnp.zeros((RT, FH), jnp.float32)
  zacc16 = jnp.zeros((RT, 16), jnp.float32)
  ones = jnp.ones((GW, 16), jnp.float32)

  cnt = _sc_counts(dst2d, zacc16, ones)

  agg1 = _sc_agg_col(x.reshape(2 * N, FH), src_off, dst2d, zacc64)

  b1r = b1.reshape(1, F)
  h1 = pl.pallas_call(
      _tc_layer1,
      grid=_GRID,
      in_specs=[_part_spec(FH), _part_spec(16), _row_spec(F),
                _full((F, F)), _full((F, F)), _full((1, F))],
      out_specs=_row_spec(F),
      out_shape=jax.ShapeDtypeStruct((N, F), jnp.float32),
  )(agg1, cnt, x, W1_l, W1_r, b1r)

  agg2 = _sc_agg_col(h1.reshape(2 * N, FH), src_off, dst2d, zacc64)

  b2r = b2.reshape(1, H2)
  w3l_pad = jnp.pad(W3_l, ((0, 0), (0, 14)))
  h2, z = pl.pallas_call(
      _tc_layer2,
      grid=_GRID,
      in_specs=[_part_spec(FH), _part_spec(16), _row_spec(F),
                _full((F, H2)), _full((F, H2)), _full((1, H2)),
                _full((H2, 16))],
      out_specs=[_row_spec(H2), _row_spec(16)],
      out_shape=[jax.ShapeDtypeStruct((N, H2), jnp.float32),
                 jax.ShapeDtypeStruct((N, 16), jnp.float32)],
  )(agg2, cnt, h1, W2_l, W2_r, b2r, w3l_pad)

  part3 = _sc_agg_16(z, src2d, dst2d, zacc16)

  w3r_pad = jnp.pad(W3_r, ((0, 0), (0, 14)))
  b3_pad = jnp.pad(b3, (0, 14)).reshape(1, 16)
  out = pl.pallas_call(
      _tc_layer3,
      grid=_GRID,
      in_specs=[_part_spec(16), _part_spec(16), _row_spec(H2),
                _full((H2, 16)), _full((1, 16))],
      out_specs=pl.BlockSpec((BR, 2), lambda i: (i, 0)),
      out_shape=jax.ShapeDtypeStruct((N, 2), jnp.float32),
  )(part3, cnt, h2, w3r_pad, b3_pad)
  return out


# speedup vs baseline: 1.5499x; 1.1908x over previous
import jax
import jax.numpy as jnp
from jax import lax
from jax.experimental import pallas as pl
from jax.experimental.pallas import tpu as pltpu
from jax.experimental.pallas import tpu_sc as plsc

N = 10000
F = 128
FH = 64
H2 = 256

NC = 2
NS = 16
NW = NC * NS

GW = 128
G_ALL = 2560
E_PAD = G_ALL * GW
GT = G_ALL // NS
GE = G_ALL // NW
N_PAD = 10240
RT = N_PAD // NS
ZT = N // NS

_MESH = plsc.VectorSubcoreMesh(core_axis_name="c", subcore_axis_name="s",
                               num_cores=NC, num_subcores=NS)


def _colsplit_body(x2_hbm, src_hbm, dst_hbm, zacc_hbm, out_hbm,
                   src_v, dst_v, rows0, rows1, acc_sh, sem0, sem1):
  c = lax.axis_index("c")
  s = lax.axis_index("s")

  pltpu.sync_copy(zacc_hbm, acc_sh.at[pl.ds(s * RT, RT)])
  pltpu.sync_copy(src_hbm.at[c].at[pl.ds(s * GT, GT)], src_v)
  pltpu.sync_copy(dst_hbm.at[pl.ds(s * GT, GT)], dst_v)
  plsc.subcore_barrier()

  rows = (rows0, rows1)
  sems = (sem0, sem1)
  pltpu.async_copy(x2_hbm.at[src_v.at[0]], rows[0], sems[0])

  @pl.loop(0, GT, step=2)
  def _(g0):
    for b in range(2):
      g = g0 + b

      @pl.when(g + 1 < GT)
      def _():
        pltpu.async_copy(x2_hbm.at[src_v.at[g + 1]], rows[1 - b],
                         sems[1 - b])

      pltpu.make_async_copy(x2_hbm.at[src_v.at[g]], rows[b], sems[b]).wait()
      pltpu.sync_copy(rows[b], acc_sh.at[dst_v.at[g]], add=True)

  plsc.subcore_barrier()
  pltpu.sync_copy(acc_sh.at[pl.ds(s * RT, RT)],
                  out_hbm.at[c].at[pl.ds(s * RT, RT)])


_sc_agg_col = pl.kernel(
    _colsplit_body,
    out_type=jax.ShapeDtypeStruct((NC, N_PAD, FH), jnp.float32),
    mesh=_MESH,
    compiler_params=pltpu.CompilerParams(use_tc_tiling_on_sc=False),
    scratch_types=[
        pltpu.VMEM((GT, GW), jnp.int32),
        pltpu.VMEM((GT, GW), jnp.int32),
        pltpu.VMEM((GW, FH), jnp.float32),
        pltpu.VMEM((GW, FH), jnp.float32),
        pltpu.VMEM_SHARED((N_PAD, FH), jnp.float32),
        pltpu.SemaphoreType.DMA,
        pltpu.SemaphoreType.DMA,
    ],
)


def _agg16_body(z_hbm, src_hbm, dst_hbm, zacc_hbm, out_hbm,
                src_v, dst_v, rows0, rows1, z_sh, acc_sh, sem0, sem1):
  c = lax.axis_index("c")
  s = lax.axis_index("s")
  wid = c * NS + s

  pltpu.sync_copy(z_hbm.at[pl.ds(s * ZT, ZT)], z_sh.at[pl.ds(s * ZT, ZT)])
  pltpu.sync_copy(zacc_hbm, acc_sh.at[pl.ds(s * RT, RT)])
  pltpu.sync_copy(src_hbm.at[pl.ds(wid * GE, GE)], src_v)
  pltpu.sync_copy(dst_hbm.at[pl.ds(wid * GE, GE)], dst_v)
  plsc.subcore_barrier()

  rows = (rows0, rows1)
  sems = (sem0, sem1)
  pltpu.async_copy(z_sh.at[src_v.at[0]], rows[0], sems[0])

  @pl.loop(0, GE, step=2)
  def _(g0):
    for b in range(2):
      g = g0 + b

      @pl.when(g + 1 < GE)
      def _():
        pltpu.async_copy(z_sh.at[src_v.at[g + 1]], rows[1 - b],
                         sems[1 - b])

      pltpu.make_async_copy(z_sh.at[src_v.at[g]], rows[b], sems[b]).wait()
      pltpu.sync_copy(rows[b], acc_sh.at[dst_v.at[g]], add=True)

  plsc.subcore_barrier()
  pltpu.sync_copy(acc_sh.at[pl.ds(s * RT, RT)],
                  out_hbm.at[c].at[pl.ds(s * RT, RT)])


_sc_agg_16 = pl.kernel(
    _agg16_body,
    out_type=jax.ShapeDtypeStruct((NC, N_PAD, 16), jnp.float32),
    mesh=_MESH,
    compiler_params=pltpu.CompilerParams(use_tc_tiling_on_sc=False),
    scratch_types=[
        pltpu.VMEM((GE, GW), jnp.int32),
        pltpu.VMEM((GE, GW), jnp.int32),
        pltpu.VMEM((GW, 16), jnp.float32),
        pltpu.VMEM((GW, 16), jnp.float32),
        pltpu.VMEM_SHARED((N, 16), jnp.float32),
        pltpu.VMEM_SHARED((N_PAD, 16), jnp.float32),
        pltpu.SemaphoreType.DMA,
        pltpu.SemaphoreType.DMA,
    ],
)


def _counts_body(dst_hbm, zacc_hbm, ones_hbm, out_hbm,
                 dst_v, ones_v, cnt_sh):
  c = lax.axis_index("c")
  s = lax.axis_index("s")
  wid = c * NS + s

  pltpu.sync_copy(zacc_hbm, cnt_sh.at[pl.ds(s * RT, RT)])
  pltpu.sync_copy(dst_hbm.at[pl.ds(wid * GE, GE)], dst_v)
  pltpu.sync_copy(ones_hbm, ones_v)
  plsc.subcore_barrier()

  @pl.loop(0, GE)
  def _(g):
    pltpu.sync_copy(ones_v, cnt_sh.at[dst_v.at[g]], add=True)

  plsc.subcore_barrier()
  pltpu.sync_copy(cnt_sh.at[pl.ds(s * RT, RT)],
                  out_hbm.at[c].at[pl.ds(s * RT, RT)])


_sc_counts = pl.kernel(
    _counts_body,
    out_type=jax.ShapeDtypeStruct((NC, N_PAD, 16), jnp.float32),
    mesh=_MESH,
    compiler_params=pltpu.CompilerParams(use_tc_tiling_on_sc=False),
    scratch_types=[
        pltpu.VMEM((GE, GW), jnp.int32),
        pltpu.VMEM((GW, 16), jnp.float32),
        pltpu.VMEM_SHARED((N_PAD, 16), jnp.float32),
    ],
)


BR = 400


def _mean(agg, cnt_ref):
  deg = (cnt_ref[0] + cnt_ref[1])[:, 0:1]
  return agg * (1.0 / jnp.maximum(deg, 1.0))


def _halves(ref):
  return jnp.concatenate([ref[0], ref[1]], axis=1)


def _tc_layer1(agg_ref, cnt_ref, x_ref, wl_ref, wr_ref, b_ref, o_ref):
  mean = _mean(_halves(agg_ref), cnt_ref)
  h = jnp.dot(mean, wl_ref[...], preferred_element_type=jnp.float32)
  h = h + b_ref[...] + jnp.dot(x_ref[...], wr_ref[...],
                               preferred_element_type=jnp.float32)
  h = jnp.maximum(h, 0.0)
  o_ref[0] = h[:, :FH]
  o_ref[1] = h[:, FH:]


def _tc_layer2(agg_ref, cnt_ref, h1_ref, wl_ref, wr_ref, b_ref, w3l_ref,
               h2_ref, z_ref):
  mean = _mean(_halves(agg_ref), cnt_ref)
  h = jnp.dot(mean, wl_ref[...], preferred_element_type=jnp.float32)
  h = h + b_ref[...] + jnp.dot(_halves(h1_ref), wr_ref[...],
                               preferred_element_type=jnp.float32)
  h2 = jnp.maximum(h, 0.0)
  h2_ref[...] = h2
  z_ref[...] = jnp.dot(h2, w3l_ref[...], preferred_element_type=jnp.float32)


def _tc_layer3(part_ref, cnt_ref, h2_ref, w3r_ref, b3_ref, o_ref):
  agg = part_ref[0] + part_ref[1]
  mean = _mean(agg, cnt_ref)
  r = jnp.dot(h2_ref[...], w3r_ref[...], preferred_element_type=jnp.float32)
  logits = (mean + r + b3_ref[...])[:, 0:2]
  m = jnp.max(logits, axis=1, keepdims=True)
  lse = m + jnp.log(jnp.sum(jnp.exp(logits - m), axis=1, keepdims=True))
  o_ref[...] = logits - lse


def _row_spec(d):
  return pl.BlockSpec((BR, d), lambda i: (i, 0))


def _part_spec(d):
  return pl.BlockSpec((NC, BR, d), lambda i: (0, i, 0))


def _full(shape):
  return pl.BlockSpec(shape, lambda i: tuple(0 for _ in shape))


_GRID = (N // BR,)


def kernel(x, edge_index, W1_l, W1_r, b1, W2_l, W2_r, b2, W3_l, W3_r, b3):
  src = edge_index[0]
  dst = edge_index[1]
  npad = E_PAD - src.shape[0]
  src_p = jnp.concatenate([src, jnp.zeros((npad,), jnp.int32)])
  dst_p = jnp.concatenate(
      [dst, N + (jnp.arange(npad, dtype=jnp.int32) % (N_PAD - N))])
  src2d = src_p.reshape(G_ALL, GW)
  dst2d = dst_p.reshape(G_ALL, GW)
  src_off = jnp.stack([src2d, src2d + N])

  zacc16 = jnp.zeros((RT, 16), jnp.float32)
  ones = jnp.ones((GW, 16), jnp.float32)

  cnt = _sc_counts(dst2d, zacc16, ones)

  zacc64 = jnp.broadcast_to(jnp.minimum(cnt[0, 0:1, 0:1], 0.0), (RT, FH))

  x_flat = jnp.concatenate([x[:, :FH], x[:, FH:]], axis=0)
  agg1 = _sc_agg_col(x_flat, src_off, dst2d, zacc64)

  b1r = b1.reshape(1, F)
  h1 = pl.pallas_call(
      _tc_layer1,
      grid=_GRID,
      in_specs=[_part_spec(FH), _part_spec(16), _row_spec(F),
                _full((F, F)), _full((F, F)), _full((1, F))],
      out_specs=pl.BlockSpec((2, BR, FH), lambda i: (0, i, 0)),
      out_shape=jax.ShapeDtypeStruct((2, N, FH), jnp.float32),
  )(agg1, cnt, x, W1_l, W1_r, b1r)

  agg2 = _sc_agg_col(h1.reshape(2 * N, FH), src_off, dst2d, zacc64)

  b2r = b2.reshape(1, H2)
  w3l_pad = jnp.pad(W3_l, ((0, 0), (0, 14)))
  h2, z = pl.pallas_call(
      _tc_layer2,
      grid=_GRID,
      in_specs=[_part_spec(FH), _part_spec(16), _part_spec(FH),
                _full((F, H2)), _full((F, H2)), _full((1, H2)),
                _full((H2, 16))],
      out_specs=[_row_spec(H2), _row_spec(16)],
      out_shape=[jax.ShapeDtypeStruct((N, H2), jnp.float32),
                 jax.ShapeDtypeStruct((N, 16), jnp.float32)],
  )(agg2, cnt, h1, W2_l, W2_r, b2r, w3l_pad)

  part3 = _sc_agg_16(z, src2d, dst2d, zacc16)

  w3r_pad = jnp.pad(W3_r, ((0, 0), (0, 14)))
  b3_pad = jnp.pad(b3, (0, 14)).reshape(1, 16)
  out = pl.pallas_call(
      _tc_layer3,
      grid=_GRID,
      in_specs=[_part_spec(16), _part_spec(16), _row_spec(H2),
                _full((H2, 16)), _full((1, 16))],
      out_specs=pl.BlockSpec((BR, 2), lambda i: (i, 0)),
      out_shape=jax.ShapeDtypeStruct((N, 2), jnp.float32),
  )(part3, cnt, h2, w3r_pad, b3_pad)
  return out


# speedup vs baseline: 1.5698x; 1.0129x over previous
import jax
import jax.numpy as jnp
from jax import lax
from jax.experimental import pallas as pl
from jax.experimental.pallas import tpu as pltpu
from jax.experimental.pallas import tpu_sc as plsc

N = 10000
F = 128
FH = 64
H2 = 256

NC = 2
NS = 16
NW = NC * NS

GW = 128
G_ALL = 2560
E_PAD = G_ALL * GW
GT = G_ALL // NS
GE = G_ALL // NW
N_PAD = 10240
RT = N_PAD // NS
ZT = N // NS

_MESH = plsc.VectorSubcoreMesh(core_axis_name="c", subcore_axis_name="s",
                               num_cores=NC, num_subcores=NS)


def _colsplit_body(x2_hbm, src_hbm, dst_hbm, zacc_hbm, out_hbm,
                   src_v, dst_v, rows0, rows1, acc_sh, sem0, sem1):
  c = lax.axis_index("c")
  s = lax.axis_index("s")

  pltpu.sync_copy(zacc_hbm, acc_sh.at[pl.ds(s * RT, RT)])
  pltpu.sync_copy(src_hbm.at[c].at[pl.ds(s * GT, GT)], src_v)
  pltpu.sync_copy(dst_hbm.at[pl.ds(s * GT, GT)], dst_v)
  plsc.subcore_barrier()

  rows = (rows0, rows1)
  sems = (sem0, sem1)
  pltpu.async_copy(x2_hbm.at[src_v.at[0]], rows[0], sems[0])

  @pl.loop(0, GT, step=2)
  def _(g0):
    for b in range(2):
      g = g0 + b

      @pl.when(g + 1 < GT)
      def _():
        pltpu.async_copy(x2_hbm.at[src_v.at[g + 1]], rows[1 - b],
                         sems[1 - b])

      pltpu.make_async_copy(x2_hbm.at[src_v.at[g]], rows[b], sems[b]).wait()
      pltpu.sync_copy(rows[b], acc_sh.at[dst_v.at[g]], add=True)

  plsc.subcore_barrier()
  pltpu.sync_copy(acc_sh.at[pl.ds(s * RT, RT)],
                  out_hbm.at[c].at[pl.ds(s * RT, RT)])


_sc_agg_col = pl.kernel(
    _colsplit_body,
    out_type=jax.ShapeDtypeStruct((NC, N_PAD, FH), jnp.float32),
    mesh=_MESH,
    compiler_params=pltpu.CompilerParams(use_tc_tiling_on_sc=False),
    scratch_types=[
        pltpu.VMEM((GT, GW), jnp.int32),
        pltpu.VMEM((GT, GW), jnp.int32),
        pltpu.VMEM((GW, FH), jnp.float32),
        pltpu.VMEM((GW, FH), jnp.float32),
        pltpu.VMEM_SHARED((N_PAD, FH), jnp.float32),
        pltpu.SemaphoreType.DMA,
        pltpu.SemaphoreType.DMA,
    ],
)


def _agg16_body(z_hbm, src_hbm, dst_hbm, zacc_hbm, out_hbm,
                src_v, dst_v, rows0, rows1, z_sh, acc_sh, sem0, sem1):
  c = lax.axis_index("c")
  s = lax.axis_index("s")
  wid = c * NS + s

  pltpu.sync_copy(z_hbm.at[pl.ds(s * ZT, ZT)], z_sh.at[pl.ds(s * ZT, ZT)])
  pltpu.sync_copy(zacc_hbm, acc_sh.at[pl.ds(s * RT, RT)])
  pltpu.sync_copy(src_hbm.at[pl.ds(wid * GE, GE)], src_v)
  pltpu.sync_copy(dst_hbm.at[pl.ds(wid * GE, GE)], dst_v)
  plsc.subcore_barrier()

  rows = (rows0, rows1)
  sems = (sem0, sem1)
  pltpu.async_copy(z_sh.at[src_v.at[0]], rows[0], sems[0])

  @pl.loop(0, GE, step=2)
  def _(g0):
    for b in range(2):
      g = g0 + b

      @pl.when(g + 1 < GE)
      def _():
        pltpu.async_copy(z_sh.at[src_v.at[g + 1]], rows[1 - b],
                         sems[1 - b])

      pltpu.make_async_copy(z_sh.at[src_v.at[g]], rows[b], sems[b]).wait()
      pltpu.sync_copy(rows[b], acc_sh.at[dst_v.at[g]], add=True)

  plsc.subcore_barrier()
  pltpu.sync_copy(acc_sh.at[pl.ds(s * RT, RT)],
                  out_hbm.at[c].at[pl.ds(s * RT, RT)])


_sc_agg_16 = pl.kernel(
    _agg16_body,
    out_type=jax.ShapeDtypeStruct((NC, N_PAD, 16), jnp.float32),
    mesh=_MESH,
    compiler_params=pltpu.CompilerParams(use_tc_tiling_on_sc=False),
    scratch_types=[
        pltpu.VMEM((GE, GW), jnp.int32),
        pltpu.VMEM((GE, GW), jnp.int32),
        pltpu.VMEM((GW, 16), jnp.float32),
        pltpu.VMEM((GW, 16), jnp.float32),
        pltpu.VMEM_SHARED((N, 16), jnp.float32),
        pltpu.VMEM_SHARED((N_PAD, 16), jnp.float32),
        pltpu.SemaphoreType.DMA,
        pltpu.SemaphoreType.DMA,
    ],
)


def _counts_body(dst_hbm, zacc_hbm, ones_hbm, out_hbm,
                 dst_v, ones_v, cnt_sh):
  c = lax.axis_index("c")
  s = lax.axis_index("s")
  wid = c * NS + s

  pltpu.sync_copy(zacc_hbm, cnt_sh.at[pl.ds(s * RT, RT)])
  pltpu.sync_copy(dst_hbm.at[pl.ds(wid * GE, GE)], dst_v)
  pltpu.sync_copy(ones_hbm, ones_v)
  plsc.subcore_barrier()

  @pl.loop(0, GE)
  def _(g):
    pltpu.sync_copy(ones_v, cnt_sh.at[dst_v.at[g]], add=True)

  plsc.subcore_barrier()
  pltpu.sync_copy(cnt_sh.at[pl.ds(s * RT, RT)],
                  out_hbm.at[c].at[pl.ds(s * RT, RT)])


_sc_counts = pl.kernel(
    _counts_body,
    out_type=jax.ShapeDtypeStruct((NC, N_PAD, 16), jnp.float32),
    mesh=_MESH,
    compiler_params=pltpu.CompilerParams(use_tc_tiling_on_sc=False),
    scratch_types=[
        pltpu.VMEM((GE, GW), jnp.int32),
        pltpu.VMEM((GW, 16), jnp.float32),
        pltpu.VMEM_SHARED((N_PAD, 16), jnp.float32),
    ],
)


BR = 400


def _mean(agg, cnt_ref):
  deg = (cnt_ref[0] + cnt_ref[1])[:, 0:1]
  return agg * (1.0 / jnp.maximum(deg, 1.0))


def _halves(ref):
  return jnp.concatenate([ref[0], ref[1]], axis=1)


def _tc_layer1(agg_ref, cnt_ref, x_ref, wl_ref, wr_ref, b_ref, o_ref):
  mean = _mean(_halves(agg_ref), cnt_ref)
  h = jnp.dot(mean, wl_ref[...], preferred_element_type=jnp.float32)
  h = h + b_ref[...] + jnp.dot(x_ref[...], wr_ref[...],
                               preferred_element_type=jnp.float32)
  h = jnp.maximum(h, 0.0)
  o_ref[0, 0] = h[:, :FH]
  o_ref[0, 1] = h[:, FH:]


def _tc_layer2(agg_ref, cnt_ref, h1_ref, wl_ref, wr_ref, b_ref, w3l_ref,
               h2_ref, z_ref):
  mean = _mean(_halves(agg_ref), cnt_ref)
  h = jnp.dot(mean, wl_ref[...], preferred_element_type=jnp.float32)
  h1 = jnp.concatenate([h1_ref[0, 0], h1_ref[0, 1]], axis=1)
  h = h + b_ref[...] + jnp.dot(h1, wr_ref[...],
                               preferred_element_type=jnp.float32)
  h2 = jnp.maximum(h, 0.0)
  h2_ref[...] = h2
  z_ref[...] = jnp.dot(h2, w3l_ref[...], preferred_element_type=jnp.float32)


def _tc_layer3(part_ref, cnt_ref, h2_ref, w3r_ref, b3_ref, o_ref):
  agg = part_ref[0] + part_ref[1]
  mean = _mean(agg, cnt_ref)
  r = jnp.dot(h2_ref[...], w3r_ref[...], preferred_element_type=jnp.float32)
  logits = (mean + r + b3_ref[...])[:, 0:2]
  m = jnp.max(logits, axis=1, keepdims=True)
  lse = m + jnp.log(jnp.sum(jnp.exp(logits - m), axis=1, keepdims=True))
  o_ref[...] = logits - lse


def _row_spec(d):
  return pl.BlockSpec((BR, d), lambda i: (i, 0))


def _part_spec(d):
  return pl.BlockSpec((NC, BR, d), lambda i: (0, i, 0))


def _full(shape):
  return pl.BlockSpec(shape, lambda i: tuple(0 for _ in shape))


_GRID = (N // BR,)


def kernel(x, edge_index, W1_l, W1_r, b1, W2_l, W2_r, b2, W3_l, W3_r, b3):
  src = edge_index[0]
  dst = edge_index[1]
  npad = E_PAD - src.shape[0]
  src_p = jnp.concatenate([src, jnp.zeros((npad,), jnp.int32)])
  dst_p = jnp.concatenate(
      [dst, N + (jnp.arange(npad, dtype=jnp.int32) % (N_PAD - N))])
  src2d = src_p.reshape(G_ALL, GW)
  dst2d = dst_p.reshape(G_ALL, GW)
  base = src2d + BR * (src2d // BR)
  src_off = jnp.stack([base, base + BR])

  zacc16 = jnp.zeros((RT, 16), jnp.float32)
  ones = jnp.ones((GW, 16), jnp.float32)

  cnt = _sc_counts(dst2d, zacc16, ones)

  zacc64 = jnp.broadcast_to(jnp.minimum(cnt[0, 0:1, 0:1], 0.0), (RT, FH))

  x_flat = x.reshape(N // BR, BR, 2, FH).transpose(0, 2, 1, 3)
  agg1 = _sc_agg_col(x_flat.reshape(2 * N, FH), src_off, dst2d, zacc64)

  b1r = b1.reshape(1, F)
  h1 = pl.pallas_call(
      _tc_layer1,
      grid=_GRID,
      in_specs=[_part_spec(FH), _part_spec(16), _row_spec(F),
                _full((F, F)), _full((F, F)), _full((1, F))],
      out_specs=pl.BlockSpec((1, 2, BR, FH), lambda i: (i, 0, 0, 0)),
      out_shape=jax.ShapeDtypeStruct((N // BR, 2, BR, FH), jnp.float32),
  )(agg1, cnt, x, W1_l, W1_r, b1r)

  agg2 = _sc_agg_col(h1.reshape(2 * N, FH), src_off, dst2d, zacc64)

  b2r = b2.reshape(1, H2)
  w3l_pad = jnp.pad(W3_l, ((0, 0), (0, 14)))
  h2, z = pl.pallas_call(
      _tc_layer2,
      grid=_GRID,
      in_specs=[_part_spec(FH), _part_spec(16),
                pl.BlockSpec((1, 2, BR, FH), lambda i: (i, 0, 0, 0)),
                _full((F, H2)), _full((F, H2)), _full((1, H2)),
                _full((H2, 16))],
      out_specs=[_row_spec(H2), _row_spec(16)],
      out_shape=[jax.ShapeDtypeStruct((N, H2), jnp.float32),
                 jax.ShapeDtypeStruct((N, 16), jnp.float32)],
  )(agg2, cnt, h1, W2_l, W2_r, b2r, w3l_pad)

  part3 = _sc_agg_16(z, src2d, dst2d, zacc16)

  w3r_pad = jnp.pad(W3_r, ((0, 0), (0, 14)))
  b3_pad = jnp.pad(b3, (0, 14)).reshape(1, 16)
  out = pl.pallas_call(
      _tc_layer3,
      grid=_GRID,
      in_specs=[_part_spec(16), _part_spec(16), _row_spec(H2),
                _full((H2, 16)), _full((1, 16))],
      out_specs=pl.BlockSpec((BR, 2), lambda i: (i, 0)),
      out_shape=jax.ShapeDtypeStruct((N, 2), jnp.float32),
  )(part3, cnt, h2, w3r_pad, b3_pad)
  return out
